# baseline (speedup 1.0000x reference)
import jax
import jax.numpy as jnp
from jax import lax
from jax.experimental import pallas as pl
from jax.experimental.pallas import tpu as pltpu
from jax.experimental.pallas import tpu_sc as plsc

_NC = 2
_NS = 16
_NW = _NC * _NS
_L = 16
_CHUNK = 128


def _iota():
    return lax.iota(jnp.int32, _L)


def _repack_body(tt, t2, strip_v, prow_v, gsem, osem):
    wid = lax.axis_index("s") * _NC + lax.axis_index("c")
    v_cols = tt.shape[1]
    n_full = v_cols // 128
    tail = v_cols - n_full * 128
    per_w = n_full // _NW + 1
    start = wid * per_w

    iot = _iota()
    rowlanes = [iot + q * _L for q in range(4)]

    def load(c, b):
        return pltpu.make_async_copy(
            tt.at[:, pl.ds(pl.multiple_of(c * 128, 128), 128)],
            strip_v.at[b],
            gsem.at[b],
        )

    def store(c, b):
        return pltpu.make_async_copy(
            prow_v.at[b],
            t2.at[pl.ds(pl.multiple_of(c * 64, 64), 64)],
            osem.at[b],
        )

    def transpose(b):
        def rr_body(rr, _):
            for q in range(8):
                col = 2 * rr + (1 if q >= 4 else 0)
                vals = plsc.load_gather(
                    strip_v.at[b], [rowlanes[q % 4], jnp.full((_L,), 0, jnp.int32) + col]
                )
                prow_v[b, rr, pl.ds(q * _L, _L)] = vals
            return _

        lax.fori_loop(0, 64, rr_body, 0)

    for b in range(2):
        c = start + b

        @pl.when(c < n_full)
        def _():
            load(c, b).start()

    def slot(i, carry):
        b = lax.rem(i, 2)
        c = start + i

        @pl.when(c < n_full)
        def _():
            load(c, b).wait()

        @pl.when(jnp.logical_and(i >= 2, c - 2 < n_full))
        def _():
            store(c - 2, b).wait()

        @pl.when(c < n_full)
        def _():
            transpose(b)
            store(c, b).start()

        @pl.when(jnp.logical_and(i + 2 < per_w, c + 2 < n_full))
        def _():
            load(c + 2, b).start()

        return carry

    lax.fori_loop(0, per_w, slot, 0)

    for b in range(2):
        c = start + per_w - 2 + b

        @pl.when(c < n_full)
        def _():
            store(c, b).wait()


def _gather_body(item_idx, user_idx, t2, ttu, tailvals, out_t, outu_t,
                 idx_v, pidx_v, rows_v, oblk_v, uidx_v, ublk_v, ttu_v, tail_v,
                 gsem, osem):
    wid = lax.axis_index("s") * _NC + lax.axis_index("c")
    per_w = idx_v.shape[0]
    n_chunks = per_w // _CHUNK
    base_w = wid * per_w
    n_pair = t2.shape[0]
    tail0 = (n_pair * 2 // 128) * 128

    iot = _iota()
    outrows = [iot + q * _L for q in range(4)]

    pltpu.sync_copy(item_idx.at[pl.ds(base_w, per_w)], idx_v)

    def mk_pidx(v, _):
        x = idx_v[pl.ds(v * _L, _L)]
        pidx_v[pl.ds(v * _L, _L)] = jnp.minimum(
            lax.shift_right_logical(x, 1), (tail0 // 2) - 1
        )
        return _

    lax.fori_loop(0, per_w // _L, mk_pidx, 0)

    pltpu.sync_copy(ttu, ttu_v)
    pltpu.sync_copy(tailvals, tail_v)
    per_w_user = uidx_v.shape[0]
    pltpu.sync_copy(user_idx.at[pl.ds(wid * per_w_user, per_w_user)], uidx_v)

    def gather(j, b):
        return pltpu.make_async_copy(
            t2.at[pidx_v.at[pl.ds(j * _CHUNK, _CHUNK)]],
            rows_v.at[b],
            gsem.at[b],
        )

    def select_t(j, b):
        def one16(v, cry):
            raw = idx_v[pl.ds(j * _CHUNK + v * _L, _L)]
            sels = (raw & 1) * 64
            tails = raw - tail0
            for i in range(_L):
                kk = v * _L + i
                kvec = jnp.full((_L,), 0, jnp.int32) + kk
                is_tail = tails[i] >= 0

                @pl.when(jnp.logical_not(is_tail))
                def _():
                    for q in range(4):
                        vals = plsc.load_gather(
                            rows_v.at[b], [kvec, iot + (sels[i] + q * _L)]
                        )
                        plsc.store_scatter(oblk_v.at[b], [outrows[q], kvec], vals)

                @pl.when(is_tail)
                def _():
                    trow = jnp.full((_L,), 0, jnp.int32) + tails[i]
                    for q in range(4):
                        vals = plsc.load_gather(tail_v, [trow, iot + q * _L])
                        plsc.store_scatter(oblk_v.at[b], [outrows[q], kvec], vals)
            return cry

        lax.fori_loop(0, _CHUNK // _L, one16, 0)

    def store(j, b):
        m = wid * n_chunks + j
        return pltpu.make_async_copy(
            oblk_v.at[b],
            out_t.at[:, pl.ds(pl.multiple_of(m * _CHUNK, _CHUNK), _CHUNK)],
            osem.at[b],
        )

    for b in range(2):
        gather(b, b).start()

    def lap_body(lap, carry):
        jj = lap * 2
        for b in range(2):
            gather(jj + b, b).wait()
            select_t(jj + b, b)
            store(jj + b, b).start()
        for b in range(2):
            store(jj + b, b).wait()
            gather(jj + 2 + b, b).start()
        return carry

    lax.fori_loop(0, n_chunks // 2 - 1, lap_body, 0)

    jj = n_chunks - 2
    for b in range(2):
        gather(jj + b, b).wait()
        select_t(jj + b, b)
        store(jj + b, b).start()
    for b in range(2):
        store(jj + b, b).wait()

    def uone16(v, _):
        uis = uidx_v[pl.ds(v * _L, _L)]
        for i in range(_L):
            kk = v * _L + i
            kvec = jnp.full((_L,), 0, jnp.int32) + kk
            vals = plsc.load_gather(ttu_v, [iot, jnp.full((_L,), 0, jnp.int32) + uis[i]])
            plsc.store_scatter(ublk_v, [iot, kvec], vals)
        return _

    lax.fori_loop(0, per_w_user // _L, uone16, 0)
    pltpu.sync_copy(
        ublk_v,
        outu_t.at[:, pl.ds(pl.multiple_of(wid * per_w_user, _CHUNK), per_w_user)],
    )


def kernel(values_item_hist, values_user_cat, table_item, table_user):
    n_hist = values_item_hist.shape[0]
    n_user = values_user_cat.shape[0]
    dim_item = table_item.shape[1]
    dim_user = table_user.shape[1]
    vocab_item = table_item.shape[0]
    vocab_user = table_user.shape[0]

    per_w = n_hist // _NW
    per_w_user = n_user // _NW

    tt = table_item.T
    ttu = table_user.T
    tail0 = (vocab_item // 128) * 128
    n_tail = vocab_item - tail0
    tailvals = lax.slice(table_item, (tail0, 0), (vocab_item, dim_item))

    mesh = plsc.VectorSubcoreMesh(core_axis_name="c", subcore_axis_name="s")

    repack = pl.kernel(
        _repack_body,
        out_type=jax.ShapeDtypeStruct((vocab_item // 2, 128), jnp.float32),
        mesh=mesh,
        compiler_params=pltpu.CompilerParams(needs_layout_passes=False),
        scratch_types=[
            pltpu.VMEM((2, dim_item, 128), jnp.float32),
            pltpu.VMEM((2, 64, 128), jnp.float32),
            pltpu.SemaphoreType.DMA((2,)),
            pltpu.SemaphoreType.DMA((2,)),
        ],
    )
    t2 = repack(tt)

    gather = pl.kernel(
        _gather_body,
        out_type=(
            jax.ShapeDtypeStruct((dim_item, n_hist), jnp.float32),
            jax.ShapeDtypeStruct((dim_user, n_user), jnp.float32),
        ),
        mesh=mesh,
        compiler_params=pltpu.CompilerParams(needs_layout_passes=False),
        scratch_types=[
            pltpu.VMEM((per_w,), jnp.int32),
            pltpu.VMEM((per_w,), jnp.int32),
            pltpu.VMEM((2, _CHUNK, 128), jnp.float32),
            pltpu.VMEM((2, dim_item, _CHUNK), jnp.float32),
            pltpu.VMEM((per_w_user,), jnp.int32),
            pltpu.VMEM((dim_user, per_w_user), jnp.float32),
            pltpu.VMEM((dim_user, vocab_user), jnp.float32),
            pltpu.VMEM((n_tail, dim_item), jnp.float32),
            pltpu.SemaphoreType.DMA((2,)),
            pltpu.SemaphoreType.DMA((2,)),
        ],
    )
    out_t, outu_t = gather(values_item_hist, values_user_cat, t2, ttu, tailvals)
    return (out_t.T, outu_t.T)


# speedup vs baseline: 2.9268x; 2.9268x over previous
import jax
import jax.numpy as jnp
from jax import lax
from jax.experimental import pallas as pl
from jax.experimental.pallas import tpu as pltpu
from jax.experimental.pallas import tpu_sc as plsc

_NC = 2
_NS = 16
_NW = _NC * _NS
_L = 16
_CHUNK = 128

_TC_COLS = 2048


def _repack_tc_body(tt_ref, t2_ref):
    x = tt_ref[...]
    h = _TC_COLS // 2
    t2_ref[...] = jnp.concatenate([x[:, :h].T, x[:, h:].T], axis=1)


def _iota():
    return lax.iota(jnp.int32, _L)


def _gather_body(item_idx, user_idx, t2, ttu, out_t, outu_t,
                 idx_v, pidx_v, rows_v, oblk_v, uidx_v, ublk_v, ttu_v,
                 gsem, osem):
    wid = lax.axis_index("s") * _NC + lax.axis_index("c")
    per_w = idx_v.shape[0]
    n_chunks = per_w // _CHUNK
    base_w = wid * per_w

    iot = _iota()

    pltpu.sync_copy(item_idx.at[pl.ds(base_w, per_w)], idx_v)

    def mk_pidx(v, cry):
        x = idx_v[pl.ds(v * _L, _L)]
        pidx_v[pl.ds(v * _L, _L)] = (
            lax.shift_left(lax.shift_right_logical(x, 11), 10) + (x & 1023)
        )
        return cry

    lax.fori_loop(0, per_w // _L, mk_pidx, 0)

    pltpu.sync_copy(ttu, ttu_v)
    per_w_user = uidx_v.shape[0]
    pltpu.sync_copy(user_idx.at[pl.ds(wid * per_w_user, per_w_user)], uidx_v)

    def gather(j, b):
        return pltpu.make_async_copy(
            t2.at[pidx_v.at[pl.ds(j * _CHUNK, _CHUNK)]],
            rows_v.at[b],
            gsem.at[b],
        )

    def select_t(j, b):
        def one16(v, cry):
            kk0 = v * _L
            sels = (lax.shift_right_logical(idx_v[pl.ds(j * _CHUNK + kk0, _L)], 10) & 1) * 64
            kvec = iot + kk0

            def one_diag(j0, cry2):
                jj = lax.rem(iot + j0, 64)
                vals = plsc.load_gather(rows_v.at[b], [kvec, sels + jj])
                plsc.store_scatter(oblk_v.at[b], [jj, kvec], vals)
                return cry2

            lax.fori_loop(0, 64, one_diag, 0)
            return cry

        lax.fori_loop(0, _CHUNK // _L, one16, 0)

    def store(j, b):
        m = wid * n_chunks + j
        return pltpu.make_async_copy(
            oblk_v.at[b],
            out_t.at[:, pl.ds(pl.multiple_of(m * _CHUNK, _CHUNK), _CHUNK)],
            osem.at[b],
        )

    for b in range(2):
        gather(b, b).start()

    def lap_body(lap, carry):
        jj = lap * 2
        for b in range(2):
            gather(jj + b, b).wait()
            select_t(jj + b, b)
            store(jj + b, b).start()
        for b in range(2):
            store(jj + b, b).wait()
            gather(jj + 2 + b, b).start()
        return carry

    lax.fori_loop(0, n_chunks // 2 - 1, lap_body, 0)

    jj = n_chunks - 2
    for b in range(2):
        gather(jj + b, b).wait()
        select_t(jj + b, b)
        store(jj + b, b).start()
    for b in range(2):
        store(jj + b, b).wait()

    def uone16(v, cry):
        uis = uidx_v[pl.ds(v * _L, _L)]
        for i in range(_L):
            kk = v * _L + i
            kvec = jnp.full((_L,), 0, jnp.int32) + kk
            vals = plsc.load_gather(ttu_v, [iot, jnp.full((_L,), 0, jnp.int32) + uis[i]])
            plsc.store_scatter(ublk_v, [iot, kvec], vals)
        return cry

    lax.fori_loop(0, per_w_user // _L, uone16, 0)
    pltpu.sync_copy(
        ublk_v,
        outu_t.at[:, pl.ds(pl.multiple_of(wid * per_w_user, _CHUNK), per_w_user)],
    )


def kernel(values_item_hist, values_user_cat, table_item, table_user):
    n_hist = values_item_hist.shape[0]
    n_user = values_user_cat.shape[0]
    dim_item = table_item.shape[1]
    dim_user = table_user.shape[1]
    vocab_item = table_item.shape[0]
    vocab_user = table_user.shape[0]

    per_w = n_hist // _NW
    per_w_user = n_user // _NW

    tt = table_item.T
    ttu = table_user.T

    n_blocks = (vocab_item + _TC_COLS - 1) // _TC_COLS
    t2 = pl.pallas_call(
        _repack_tc_body,
        grid=(n_blocks,),
        in_specs=[pl.BlockSpec((dim_item, _TC_COLS), lambda i: (0, i))],
        out_specs=pl.BlockSpec((_TC_COLS // 2, 128), lambda i: (i, 0)),
        out_shape=jax.ShapeDtypeStruct((n_blocks * (_TC_COLS // 2), 128), jnp.float32),
    )(tt)

    mesh = plsc.VectorSubcoreMesh(core_axis_name="c", subcore_axis_name="s")
    gather = pl.kernel(
        _gather_body,
        out_type=(
            jax.ShapeDtypeStruct((dim_item, n_hist), jnp.float32),
            jax.ShapeDtypeStruct((dim_user, n_user), jnp.float32),
        ),
        mesh=mesh,
        compiler_params=pltpu.CompilerParams(needs_layout_passes=False),
        scratch_types=[
            pltpu.VMEM((per_w,), jnp.int32),
            pltpu.VMEM((per_w,), jnp.int32),
            pltpu.VMEM((2, _CHUNK, 128), jnp.float32),
            pltpu.VMEM((2, dim_item, _CHUNK), jnp.float32),
            pltpu.VMEM((per_w_user,), jnp.int32),
            pltpu.VMEM((dim_user, per_w_user), jnp.float32),
            pltpu.VMEM((dim_user, vocab_user), jnp.float32),
            pltpu.SemaphoreType.DMA((2,)),
            pltpu.SemaphoreType.DMA((2,)),
        ],
    )
    out_t, outu_t = gather(values_item_hist, values_user_cat, t2, ttu)
    return (out_t.T, outu_t.T)


# speedup vs baseline: 4.2375x; 1.4478x over previous
import jax
import jax.numpy as jnp
from jax import lax
from jax.experimental import pallas as pl
from jax.experimental.pallas import tpu as pltpu
from jax.experimental.pallas import tpu_sc as plsc

_NC = 2
_NS = 16
_NW = _NC * _NS
_L = 16
_CHUNK = 128

_TC_COLS = 8192


def _repack_tc_body(tt_ref, t2_ref):
    for s in range(_TC_COLS // 2048):
        x = tt_ref[:, pl.ds(s * 2048, 2048)]
        t2_ref[pl.ds(s * 1024, 1024), :] = jnp.concatenate(
            [x[:, :1024].T, x[:, 1024:].T], axis=1
        )


def _iota():
    return lax.iota(jnp.int32, _L)


def _gather_body(item_idx, user_idx, t2, ttu, out_t, outu_t,
                 idx_v, pidx_v, rows_v, oblk_v, uidx_v, ublk_v, ttu_v,
                 gsem, osem):
    wid = lax.axis_index("s") * _NC + lax.axis_index("c")
    per_w = idx_v.shape[0]
    n_chunks = per_w // _CHUNK
    base_w = wid * per_w

    iot = _iota()

    pltpu.sync_copy(item_idx.at[pl.ds(base_w, per_w)], idx_v)

    def mk_pidx(v, cry):
        x = idx_v[pl.ds(v * _L, _L)]
        pidx_v[pl.ds(v * _L, _L)] = (
            lax.shift_left(lax.shift_right_logical(x, 11), 10) + (x & 1023)
        )
        return cry

    lax.fori_loop(0, per_w // _L, mk_pidx, 0)

    pltpu.sync_copy(ttu, ttu_v)
    per_w_user = uidx_v.shape[0]
    pltpu.sync_copy(user_idx.at[pl.ds(wid * per_w_user, per_w_user)], uidx_v)

    def gather(j, b):
        return pltpu.make_async_copy(
            t2.at[pidx_v.at[pl.ds(j * _CHUNK, _CHUNK)]],
            rows_v.at[b],
            gsem.at[b],
        )

    def select_t(j, b):
        def one16(v, cry):
            kk0 = v * _L
            sels = (lax.shift_right_logical(idx_v[pl.ds(j * _CHUNK + kk0, _L)], 10) & 1) * 64
            kvec = iot + kk0

            def one_diag(d, cry2):
                for u in range(2):
                    jj = (iot + (2 * d + u)) & 63
                    vals = plsc.load_gather(rows_v.at[b], [kvec, sels + jj])
                    plsc.store_scatter(oblk_v.at[b], [jj, kvec], vals)
                return cry2

            lax.fori_loop(0, 32, one_diag, 0)
            return cry

        lax.fori_loop(0, _CHUNK // _L, one16, 0)

    def store(j, b):
        m = wid * n_chunks + j
        return pltpu.make_async_copy(
            oblk_v.at[b],
            out_t.at[:, pl.ds(pl.multiple_of(m * _CHUNK, _CHUNK), _CHUNK)],
            osem.at[b],
        )

    for b in range(2):
        gather(b, b).start()

    def lap_body(lap, carry):
        jj = lap * 2
        for b in range(2):
            gather(jj + b, b).wait()
            select_t(jj + b, b)
            store(jj + b, b).start()
        for b in range(2):
            store(jj + b, b).wait()
            gather(jj + 2 + b, b).start()
        return carry

    lax.fori_loop(0, n_chunks // 2 - 1, lap_body, 0)

    jj = n_chunks - 2
    for b in range(2):
        gather(jj + b, b).wait()
        select_t(jj + b, b)
        store(jj + b, b).start()
    for b in range(2):
        store(jj + b, b).wait()

    def uone16(v, cry):
        uis = uidx_v[pl.ds(v * _L, _L)]
        for i in range(_L):
            kk = v * _L + i
            kvec = jnp.full((_L,), 0, jnp.int32) + kk
            vals = plsc.load_gather(ttu_v, [iot, jnp.full((_L,), 0, jnp.int32) + uis[i]])
            plsc.store_scatter(ublk_v, [iot, kvec], vals)
        return cry

    lax.fori_loop(0, per_w_user // _L, uone16, 0)
    pltpu.sync_copy(
        ublk_v,
        outu_t.at[:, pl.ds(pl.multiple_of(wid * per_w_user, _CHUNK), per_w_user)],
    )


def kernel(values_item_hist, values_user_cat, table_item, table_user):
    n_hist = values_item_hist.shape[0]
    n_user = values_user_cat.shape[0]
    dim_item = table_item.shape[1]
    dim_user = table_user.shape[1]
    vocab_item = table_item.shape[0]
    vocab_user = table_user.shape[0]

    per_w = n_hist // _NW
    per_w_user = n_user // _NW

    tt = table_item.T
    ttu = table_user.T

    n_blocks = (vocab_item + _TC_COLS - 1) // _TC_COLS
    t2 = pl.pallas_call(
        _repack_tc_body,
        grid=(n_blocks,),
        in_specs=[pl.BlockSpec((dim_item, _TC_COLS), lambda i: (0, i))],
        out_specs=pl.BlockSpec((_TC_COLS // 2, 128), lambda i: (i, 0)),
        out_shape=jax.ShapeDtypeStruct((n_blocks * (_TC_COLS // 2), 128), jnp.float32),
    )(tt)

    mesh = plsc.VectorSubcoreMesh(core_axis_name="c", subcore_axis_name="s")
    gather = pl.kernel(
        _gather_body,
        out_type=(
            jax.ShapeDtypeStruct((dim_item, n_hist), jnp.float32),
            jax.ShapeDtypeStruct((dim_user, n_user), jnp.float32),
        ),
        mesh=mesh,
        compiler_params=pltpu.CompilerParams(needs_layout_passes=False),
        scratch_types=[
            pltpu.VMEM((per_w,), jnp.int32),
            pltpu.VMEM((per_w,), jnp.int32),
            pltpu.VMEM((2, _CHUNK, 128), jnp.float32),
            pltpu.VMEM((2, dim_item, _CHUNK), jnp.float32),
            pltpu.VMEM((per_w_user,), jnp.int32),
            pltpu.VMEM((dim_user, per_w_user), jnp.float32),
            pltpu.VMEM((dim_user, vocab_user), jnp.float32),
            pltpu.SemaphoreType.DMA((2,)),
            pltpu.SemaphoreType.DMA((2,)),
        ],
    )
    out_t, outu_t = gather(values_item_hist, values_user_cat, t2, ttu)
    return (out_t.T, outu_t.T)


# speedup vs baseline: 4.9404x; 1.1659x over previous
import jax
import jax.numpy as jnp
from jax import lax
from jax.experimental import pallas as pl
from jax.experimental.pallas import tpu as pltpu
from jax.experimental.pallas import tpu_sc as plsc

_NC = 2
_NS = 16
_NW = _NC * _NS
_L = 16
_CHUNK = 128

_TC_COLS = 8192


def _repack_tc_body(tt_ref, t2_ref):
    for s in range(_TC_COLS // 2048):
        x = tt_ref[:, pl.ds(s * 2048, 2048)]
        t2_ref[pl.ds(s * 1024, 1024), :] = jnp.concatenate(
            [x[:, :1024], x[:, 1024:]], axis=0
        ).T


def _iota():
    return lax.iota(jnp.int32, _L)


def _gather_body(item_idx, user_idx, t2, ttu, out_t, outu_t,
                 idx_v, pidx_v, rows_v, oblk_v, uidx_v, ublk_v, ttu_v,
                 gsem, osem):
    wid = lax.axis_index("s") * _NC + lax.axis_index("c")
    per_w = idx_v.shape[0]
    n_chunks = per_w // _CHUNK
    base_w = wid * per_w

    iot = _iota()

    pltpu.sync_copy(item_idx.at[pl.ds(base_w, per_w)], idx_v)

    def mk_pidx(v, cry):
        x = idx_v[pl.ds(v * _L, _L)]
        pidx_v[pl.ds(v * _L, _L)] = (
            lax.shift_left(lax.shift_right_logical(x, 11), 10) + (x & 1023)
        )
        return cry

    lax.fori_loop(0, per_w // _L, mk_pidx, 0)

    pltpu.sync_copy(ttu, ttu_v)
    per_w_user = uidx_v.shape[0]
    pltpu.sync_copy(user_idx.at[pl.ds(wid * per_w_user, per_w_user)], uidx_v)

    def gather(j, b):
        return pltpu.make_async_copy(
            t2.at[pidx_v.at[pl.ds(j * _CHUNK, _CHUNK)]],
            rows_v.at[b],
            gsem.at[b],
        )

    def select_t(j, b):
        def one16(v, cry):
            kk0 = v * _L
            sels = (lax.shift_right_logical(idx_v[pl.ds(j * _CHUNK + kk0, _L)], 10) & 1) * 64
            kvec = iot + kk0

            def one_diag(d, cry2):
                for u in range(2):
                    jj = (iot + (2 * d + u)) & 63
                    vals = plsc.load_gather(rows_v.at[b], [kvec, sels + jj])
                    plsc.store_scatter(oblk_v.at[b], [jj, kvec], vals)
                return cry2

            lax.fori_loop(0, 32, one_diag, 0)
            return cry

        lax.fori_loop(0, _CHUNK // _L, one16, 0)

    def store(j, b):
        m = wid * n_chunks + j
        return pltpu.make_async_copy(
            oblk_v.at[b],
            out_t.at[:, pl.ds(pl.multiple_of(m * _CHUNK, _CHUNK), _CHUNK)],
            osem.at[b],
        )

    for b in range(2):
        gather(b, b).start()

    def lap_body(lap, carry):
        jj = lap * 2
        for b in range(2):
            gather(jj + b, b).wait()
            select_t(jj + b, b)
            store(jj + b, b).start()
        for b in range(2):
            store(jj + b, b).wait()
            gather(jj + 2 + b, b).start()
        return carry

    lax.fori_loop(0, n_chunks // 2 - 1, lap_body, 0)

    jj = n_chunks - 2
    for b in range(2):
        gather(jj + b, b).wait()
        select_t(jj + b, b)
        store(jj + b, b).start()
    for b in range(2):
        store(jj + b, b).wait()

    def uone16(v, cry):
        uis = uidx_v[pl.ds(v * _L, _L)]
        for i in range(_L):
            kk = v * _L + i
            kvec = jnp.full((_L,), 0, jnp.int32) + kk
            vals = plsc.load_gather(ttu_v, [iot, jnp.full((_L,), 0, jnp.int32) + uis[i]])
            plsc.store_scatter(ublk_v, [iot, kvec], vals)
        return cry

    lax.fori_loop(0, per_w_user // _L, uone16, 0)
    pltpu.sync_copy(
        ublk_v,
        outu_t.at[:, pl.ds(pl.multiple_of(wid * per_w_user, _CHUNK), per_w_user)],
    )


def kernel(values_item_hist, values_user_cat, table_item, table_user):
    n_hist = values_item_hist.shape[0]
    n_user = values_user_cat.shape[0]
    dim_item = table_item.shape[1]
    dim_user = table_user.shape[1]
    vocab_item = table_item.shape[0]
    vocab_user = table_user.shape[0]

    per_w = n_hist // _NW
    per_w_user = n_user // _NW

    tt = table_item.T
    ttu = table_user.T

    n_blocks = (vocab_item + _TC_COLS - 1) // _TC_COLS
    t2 = pl.pallas_call(
        _repack_tc_body,
        grid=(n_blocks,),
        in_specs=[pl.BlockSpec((dim_item, _TC_COLS), lambda i: (0, i))],
        out_specs=pl.BlockSpec((_TC_COLS // 2, 128), lambda i: (i, 0)),
        out_shape=jax.ShapeDtypeStruct((n_blocks * (_TC_COLS // 2), 128), jnp.float32),
    )(tt)

    mesh = plsc.VectorSubcoreMesh(core_axis_name="c", subcore_axis_name="s")
    gather = pl.kernel(
        _gather_body,
        out_type=(
            jax.ShapeDtypeStruct((dim_item, n_hist), jnp.float32),
            jax.ShapeDtypeStruct((dim_user, n_user), jnp.float32),
        ),
        mesh=mesh,
        compiler_params=pltpu.CompilerParams(needs_layout_passes=False),
        scratch_types=[
            pltpu.VMEM((per_w,), jnp.int32),
            pltpu.VMEM((per_w,), jnp.int32),
            pltpu.VMEM((2, _CHUNK, 128), jnp.float32),
            pltpu.VMEM((2, dim_item, _CHUNK), jnp.float32),
            pltpu.VMEM((per_w_user,), jnp.int32),
            pltpu.VMEM((dim_user, per_w_user), jnp.float32),
            pltpu.VMEM((dim_user, vocab_user), jnp.float32),
            pltpu.SemaphoreType.DMA((2,)),
            pltpu.SemaphoreType.DMA((2,)),
        ],
    )
    out_t, outu_t = gather(values_item_hist, values_user_cat, t2, ttu)
    return (out_t.T, outu_t.T)


# speedup vs baseline: 5.3884x; 1.0907x over previous
import jax
import jax.numpy as jnp
from jax import lax
from jax.experimental import pallas as pl
from jax.experimental.pallas import tpu as pltpu
from jax.experimental.pallas import tpu_sc as plsc

_NC = 2
_NS = 16
_NW = _NC * _NS
_L = 16
_CHUNK = 128

_TC_COLS = 16384


def _repack_tc_body(tt_ref, t2_ref):
    for s in range(_TC_COLS // 2048):
        x = tt_ref[:, pl.ds(s * 2048, 2048)]
        t2_ref[pl.ds(s * 1024, 1024), :] = jnp.concatenate(
            [x[:, :1024], x[:, 1024:]], axis=0
        ).T


def _iota():
    return lax.iota(jnp.int32, _L)


def _gather_body(item_idx, user_idx, t2, ttu, out_t, outu_t,
                 idx_v, pidx_v, rows_v, oblk_v, uidx_v, ublk_v, ttu_v,
                 gsem, osem):
    wid = lax.axis_index("s") * _NC + lax.axis_index("c")
    per_w = idx_v.shape[0]
    n_chunks = per_w // _CHUNK
    base_w = wid * per_w

    iot = _iota()

    pltpu.sync_copy(item_idx.at[pl.ds(base_w, per_w)], idx_v)

    def mk_pidx(v, cry):
        x = idx_v[pl.ds(v * _L, _L)]
        pidx_v[pl.ds(v * _L, _L)] = (
            lax.shift_left(lax.shift_right_logical(x, 11), 10) + (x & 1023)
        )
        return cry

    lax.fori_loop(0, per_w // _L, mk_pidx, 0)

    pltpu.sync_copy(ttu, ttu_v)
    per_w_user = uidx_v.shape[0]
    pltpu.sync_copy(user_idx.at[pl.ds(wid * per_w_user, per_w_user)], uidx_v)

    def gather(j, b):
        return pltpu.make_async_copy(
            t2.at[pidx_v.at[pl.ds(j * _CHUNK, _CHUNK)]],
            rows_v.at[b],
            gsem.at[b],
        )

    def select_t(j, b):
        def one16(v, cry):
            kk0 = v * _L
            sels = (lax.shift_right_logical(idx_v[pl.ds(j * _CHUNK + kk0, _L)], 10) & 1) * 64
            kvec = iot + kk0

            def one_diag(d, cry2):
                for u in range(4):
                    jj = (iot + (4 * d + u)) & 63
                    vals = plsc.load_gather(rows_v.at[b], [kvec, sels + jj])
                    plsc.store_scatter(oblk_v.at[b], [jj, kvec], vals)
                return cry2

            lax.fori_loop(0, 16, one_diag, 0)
            return cry

        lax.fori_loop(0, _CHUNK // _L, one16, 0)

    def store(j, b):
        m = wid * n_chunks + j
        return pltpu.make_async_copy(
            oblk_v.at[b],
            out_t.at[:, pl.ds(pl.multiple_of(m * _CHUNK, _CHUNK), _CHUNK)],
            osem.at[b],
        )

    for b in range(2):
        gather(b, b).start()

    def lap_body(lap, carry):
        jj = lap * 2
        for b in range(2):
            gather(jj + b, b).wait()
            select_t(jj + b, b)
            store(jj + b, b).start()
        for b in range(2):
            store(jj + b, b).wait()
            gather(jj + 2 + b, b).start()
        return carry

    lax.fori_loop(0, n_chunks // 2 - 1, lap_body, 0)

    jj = n_chunks - 2
    for b in range(2):
        gather(jj + b, b).wait()
        select_t(jj + b, b)
        store(jj + b, b).start()
    for b in range(2):
        store(jj + b, b).wait()

    def uone16(v, cry):
        uis = uidx_v[pl.ds(v * _L, _L)]
        for i in range(_L):
            kk = v * _L + i
            kvec = jnp.full((_L,), 0, jnp.int32) + kk
            vals = plsc.load_gather(ttu_v, [iot, jnp.full((_L,), 0, jnp.int32) + uis[i]])
            plsc.store_scatter(ublk_v, [iot, kvec], vals)
        return cry

    lax.fori_loop(0, per_w_user // _L, uone16, 0)
    pltpu.sync_copy(
        ublk_v,
        outu_t.at[:, pl.ds(pl.multiple_of(wid * per_w_user, _CHUNK), per_w_user)],
    )


def kernel(values_item_hist, values_user_cat, table_item, table_user):
    n_hist = values_item_hist.shape[0]
    n_user = values_user_cat.shape[0]
    dim_item = table_item.shape[1]
    dim_user = table_user.shape[1]
    vocab_item = table_item.shape[0]
    vocab_user = table_user.shape[0]

    per_w = n_hist // _NW
    per_w_user = n_user // _NW

    tt = table_item.T
    ttu = table_user.T

    n_blocks = (vocab_item + _TC_COLS - 1) // _TC_COLS
    t2 = pl.pallas_call(
        _repack_tc_body,
        grid=(n_blocks,),
        in_specs=[pl.BlockSpec((dim_item, _TC_COLS), lambda i: (0, i))],
        out_specs=pl.BlockSpec((_TC_COLS // 2, 128), lambda i: (i, 0)),
        out_shape=jax.ShapeDtypeStruct((n_blocks * (_TC_COLS // 2), 128), jnp.float32),
    )(tt)

    mesh = plsc.VectorSubcoreMesh(core_axis_name="c", subcore_axis_name="s")
    gather = pl.kernel(
        _gather_body,
        out_type=(
            jax.ShapeDtypeStruct((dim_item, n_hist), jnp.float32),
            jax.ShapeDtypeStruct((dim_user, n_user), jnp.float32),
        ),
        mesh=mesh,
        compiler_params=pltpu.CompilerParams(needs_layout_passes=False),
        scratch_types=[
            pltpu.VMEM((per_w,), jnp.int32),
            pltpu.VMEM((per_w,), jnp.int32),
            pltpu.VMEM((2, _CHUNK, 128), jnp.float32),
            pltpu.VMEM((2, dim_item, _CHUNK), jnp.float32),
            pltpu.VMEM((per_w_user,), jnp.int32),
            pltpu.VMEM((dim_user, per_w_user), jnp.float32),
            pltpu.VMEM((dim_user, vocab_user), jnp.float32),
            pltpu.SemaphoreType.DMA((2,)),
            pltpu.SemaphoreType.DMA((2,)),
        ],
    )
    out_t, outu_t = gather(values_item_hist, values_user_cat, t2, ttu)
    return (out_t.T, outu_t.T)


# speedup vs baseline: 6.0967x; 1.1314x over previous
import jax
import jax.numpy as jnp
from jax import lax
from jax.experimental import pallas as pl
from jax.experimental.pallas import tpu as pltpu
from jax.experimental.pallas import tpu_sc as plsc

_NC = 2
_NS = 16
_NW = _NC * _NS
_L = 16
_CHUNK = 128
_NBUF = 4

_TC_COLS = 16384


def _repack_tc_body(tt_ref, t2_ref):
    for s in range(_TC_COLS // 2048):
        x = tt_ref[:, pl.ds(s * 2048, 2048)]
        t2_ref[pl.ds(s * 1024, 1024), :] = jnp.concatenate(
            [x[:, :1024], x[:, 1024:]], axis=0
        ).T


def _iota():
    return lax.iota(jnp.int32, _L)


def _gather_body(item_idx, user_idx, t2, ttu, out_t, outu_t,
                 idx_v, pidx_v, rows_v, oblk_v, uidx_v, ublk_v, ttu_v,
                 gsem, osem):
    wid = lax.axis_index("s") * _NC + lax.axis_index("c")
    per_w = idx_v.shape[0]
    n_chunks = per_w // _CHUNK
    base_w = wid * per_w

    iot = _iota()

    pltpu.sync_copy(item_idx.at[pl.ds(base_w, per_w)], idx_v)

    def mk_pidx(v, cry):
        x = idx_v[pl.ds(v * _L, _L)]
        pidx_v[pl.ds(v * _L, _L)] = (
            lax.shift_left(lax.shift_right_logical(x, 11), 10) + (x & 1023)
        )
        return cry

    lax.fori_loop(0, per_w // _L, mk_pidx, 0)

    pltpu.sync_copy(ttu, ttu_v)
    per_w_user = uidx_v.shape[0]
    pltpu.sync_copy(user_idx.at[pl.ds(wid * per_w_user, per_w_user)], uidx_v)

    def gather(j, b):
        return pltpu.make_async_copy(
            t2.at[pidx_v.at[pl.ds(j * _CHUNK, _CHUNK)]],
            rows_v.at[b],
            gsem.at[b],
        )

    def select_t(j, b):
        def one16(v, cry):
            kk0 = v * _L
            sels = (lax.shift_right_logical(idx_v[pl.ds(j * _CHUNK + kk0, _L)], 10) & 1) * 64
            kvec = iot + kk0

            def one_diag(d, cry2):
                for u in range(4):
                    jj = (iot + (4 * d + u)) & 63
                    vals = plsc.load_gather(rows_v.at[b], [kvec, sels + jj])
                    plsc.store_scatter(oblk_v.at[b], [jj, kvec], vals)
                return cry2

            lax.fori_loop(0, 16, one_diag, 0)
            return cry

        lax.fori_loop(0, _CHUNK // _L, one16, 0)

    def store(j, b):
        m = wid * n_chunks + j
        return pltpu.make_async_copy(
            oblk_v.at[b],
            out_t.at[:, pl.ds(pl.multiple_of(m * _CHUNK, _CHUNK), _CHUNK)],
            osem.at[b],
        )

    for b in range(_NBUF):
        gather(b, b).start()

    def lap_body(lap, carry):
        jj = lap * _NBUF
        for b in range(_NBUF):
            j = jj + b
            gather(j, b).wait()

            @pl.when(lap > 0)
            def _():
                store(j - _NBUF, b).wait()

            select_t(j, b)
            store(j, b).start()

            @pl.when(j + _NBUF < n_chunks)
            def _():
                gather(j + _NBUF, b).start()

        return carry

    n_laps = n_chunks // _NBUF
    lax.fori_loop(0, n_laps, lap_body, 0)
    for b in range(n_chunks - n_laps * _NBUF):
        j = n_laps * _NBUF + b
        gather(j, b).wait()
        store(j - _NBUF, b).wait()
        select_t(j, b)
        store(j, b).start()
    for j in range(n_chunks - _NBUF, n_chunks):
        store(j, j % _NBUF).wait()

    def uone16(v, cry):
        uis = uidx_v[pl.ds(v * _L, _L)]
        for i in range(_L):
            kk = v * _L + i
            kvec = jnp.full((_L,), 0, jnp.int32) + kk
            vals = plsc.load_gather(ttu_v, [iot, jnp.full((_L,), 0, jnp.int32) + uis[i]])
            plsc.store_scatter(ublk_v, [iot, kvec], vals)
        return cry

    lax.fori_loop(0, per_w_user // _L, uone16, 0)
    pltpu.sync_copy(
        ublk_v,
        outu_t.at[:, pl.ds(pl.multiple_of(wid * per_w_user, _CHUNK), per_w_user)],
    )


def kernel(values_item_hist, values_user_cat, table_item, table_user):
    n_hist = values_item_hist.shape[0]
    n_user = values_user_cat.shape[0]
    dim_item = table_item.shape[1]
    dim_user = table_user.shape[1]
    vocab_item = table_item.shape[0]
    vocab_user = table_user.shape[0]

    per_w = n_hist // _NW
    per_w_user = n_user // _NW

    tt = table_item.T
    ttu = table_user.T

    n_blocks = (vocab_item + _TC_COLS - 1) // _TC_COLS
    t2 = pl.pallas_call(
        _repack_tc_body,
        grid=(n_blocks,),
        in_specs=[pl.BlockSpec((dim_item, _TC_COLS), lambda i: (0, i))],
        out_specs=pl.BlockSpec((_TC_COLS // 2, 128), lambda i: (i, 0)),
        out_shape=jax.ShapeDtypeStruct((n_blocks * (_TC_COLS // 2), 128), jnp.float32),
    )(tt)

    mesh = plsc.VectorSubcoreMesh(core_axis_name="c", subcore_axis_name="s")
    gather = pl.kernel(
        _gather_body,
        out_type=(
            jax.ShapeDtypeStruct((dim_item, n_hist), jnp.float32),
            jax.ShapeDtypeStruct((dim_user, n_user), jnp.float32),
        ),
        mesh=mesh,
        compiler_params=pltpu.CompilerParams(needs_layout_passes=False),
        scratch_types=[
            pltpu.VMEM((per_w,), jnp.int32),
            pltpu.VMEM((per_w,), jnp.int32),
            pltpu.VMEM((_NBUF, _CHUNK, 128), jnp.float32),
            pltpu.VMEM((_NBUF, dim_item, _CHUNK), jnp.float32),
            pltpu.VMEM((per_w_user,), jnp.int32),
            pltpu.VMEM((dim_user, per_w_user), jnp.float32),
            pltpu.VMEM((dim_user, vocab_user), jnp.float32),
            pltpu.SemaphoreType.DMA((_NBUF,)),
            pltpu.SemaphoreType.DMA((_NBUF,)),
        ],
    )
    out_t, outu_t = gather(values_item_hist, values_user_cat, t2, ttu)
    return (out_t.T, outu_t.T)
